# Initial kernel scaffold; baseline (speedup 1.0000x reference)
import jax
import jax.numpy as jnp
from jax.experimental import pallas as pl


def kernel(x, edge_index, edge_attr, batch, Wn, bn, We, be, Wc1, bc1, Wc2, bc2, Wc3, bc3, Wk1, bk1, Wk2, bk2):
    raise NotImplementedError("write your pallas kernel here")


# speedup vs baseline: 10.9656x; 10.9656x over previous
import jax
import jax.numpy as jnp
from jax import lax
from jax.experimental import pallas as pl
from jax.experimental.pallas import tpu as pltpu
from jax.experimental.pallas import tpu_sc as plsc

N = 10000
E = 320000
DF = 128
DE = 16
H = 64
G = 64

NC = 2
NS = 16
NW = NC * NS
EPW = E // NW
CK = 128
NCH = 80
PAD = NCH * CK - EPW
ND = N + 16
FE = EPW // CK
TAIL = EPW - FE * CK
RPS = 624
BR = 1000
NB = N // BR

_mesh = plsc.VectorSubcoreMesh(core_axis_name="c", subcore_axis_name="s")
_f32 = jnp.float32
_sc_params = pltpu.CompilerParams(use_tc_tiling_on_sc=False)


def _zero_rows(zbuf, rows, width):
    zv = jnp.zeros((16,), _f32)

    def _z(i, _):
        for j in range(width // 16):
            zbuf[i, pl.ds(j * 16, 16)] = zv
        return 0

    lax.fori_loop(0, rows, _z, 0)


def _init_acc(tables, zbuf, sid):
    for t in tables:
        for k in range(RPS // 208):
            pltpu.sync_copy(zbuf.at[pl.ds(0, 208)],
                            t.at[pl.ds(sid * RPS + k * 208, 208)])

    @pl.when(sid == NS - 1)
    def _():
        for t in tables:
            pltpu.sync_copy(zbuf.at[pl.ds(0, 16)], t.at[pl.ds(NS * RPS, 16)])


def _copy_out(acc, out, cid, sid):
    sl = pl.ds(sid * RPS, RPS)
    pltpu.sync_copy(acc.at[sl], out.at[cid, sl])

    @pl.when(sid == NS - 1)
    def _():
        tl = pl.ds(NS * RPS, 16)
        pltpu.sync_copy(acc.at[tl], out.at[cid, tl])


def _edge_body(ea_hbm, row_hbm, col_hbm,
               oaR, oaC, ocR, ocC,
               idxR, idxC, eb0, eb1, ones, zbuf,
               accR, accC, cntR, cntC, semA, semB):
    cid = lax.axis_index("c")
    sid = lax.axis_index("s")
    w = cid * NS + sid
    base = w * EPW

    _zero_rows(zbuf, 208, DE)
    _init_acc((accR, accC, cntR, cntC), zbuf, sid)

    ov = jnp.ones((16,), _f32)

    def _o(i, _):
        ones[i, :] = ov
        return 0

    lax.fori_loop(0, CK, _o, 0)

    pltpu.sync_copy(row_hbm.at[w], idxR)
    pltpu.sync_copy(col_hbm.at[w], idxC)
    plsc.subcore_barrier()

    def _scat(buf, c):
        pltpu.sync_copy(buf, accR.at[idxR.at[c]], add=True)
        pltpu.sync_copy(buf, accC.at[idxC.at[c]], add=True)
        pltpu.sync_copy(ones, cntR.at[idxR.at[c]], add=True)
        pltpu.sync_copy(ones, cntC.at[idxC.at[c]], add=True)

    NP = FE // 2
    pltpu.async_copy(ea_hbm.at[pl.ds(base, CK)], eb0, semA)

    def _body(p, _):
        i0 = p * 2
        i1 = i0 + 1
        pltpu.make_async_copy(ea_hbm.at[pl.ds(base + i0 * CK, CK)], eb0, semA).wait()
        pltpu.async_copy(ea_hbm.at[pl.ds(base + i1 * CK, CK)], eb1, semB)
        _scat(eb0, i0)
        pltpu.make_async_copy(ea_hbm.at[pl.ds(base + i1 * CK, CK)], eb1, semB).wait()

        @pl.when(p + 1 < NP)
        def _():
            pltpu.async_copy(ea_hbm.at[pl.ds(base + (i0 + 2) * CK, CK)], eb0, semA)

        _scat(eb1, i1)
        return 0

    lax.fori_loop(0, NP, _body, 0)

    pltpu.sync_copy(ea_hbm.at[pl.ds(base + FE * CK, TAIL)], eb0.at[pl.ds(0, TAIL)])
    _scat(eb0, FE)

    plsc.subcore_barrier()
    _copy_out(accR, oaR, cid, sid)
    _copy_out(accC, oaC, cid, sid)
    _copy_out(cntR, ocR, cid, sid)
    _copy_out(cntC, ocC, cid, sid)


_edge_pass = pl.kernel(
    _edge_body,
    out_type=[jax.ShapeDtypeStruct((NC, N, DE), _f32) for _ in range(4)],
    mesh=_mesh,
    scratch_types=[
        pltpu.VMEM((NCH, CK), jnp.int32),
        pltpu.VMEM((NCH, CK), jnp.int32),
        pltpu.VMEM((CK, DE), _f32),
        pltpu.VMEM((CK, DE), _f32),
        pltpu.VMEM((CK, DE), _f32),
        pltpu.VMEM((208, DE), _f32),
        pltpu.VMEM_SHARED((ND, DE), _f32),
        pltpu.VMEM_SHARED((ND, DE), _f32),
        pltpu.VMEM_SHARED((ND, DE), _f32),
        pltpu.VMEM_SHARED((ND, DE), _f32),
        pltpu.SemaphoreType.DMA,
        pltpu.SemaphoreType.DMA,
    ],
    compiler_params=_sc_params,
)


def _conv_body(u_hbm, row_hbm, col_hbm, oacc,
               idxR, idxC, g0, g1, zbuf, acc, semA, semB):
    cid = lax.axis_index("c")
    sid = lax.axis_index("s")
    w = cid * NS + sid

    _zero_rows(zbuf, 208, H)
    _init_acc((acc,), zbuf, sid)

    pltpu.sync_copy(row_hbm.at[w], idxR)
    pltpu.sync_copy(col_hbm.at[w], idxC)
    plsc.subcore_barrier()

    NP = NCH // 2
    pltpu.async_copy(u_hbm.at[idxR.at[0]], g0, semA)

    def _body(p, _):
        i0 = p * 2
        i1 = i0 + 1
        pltpu.make_async_copy(u_hbm.at[idxR.at[i0]], g0, semA).wait()
        pltpu.async_copy(u_hbm.at[idxR.at[i1]], g1, semB)
        pltpu.sync_copy(g0, acc.at[idxC.at[i0]], add=True)
        pltpu.make_async_copy(u_hbm.at[idxR.at[i1]], g1, semB).wait()

        @pl.when(p + 1 < NP)
        def _():
            pltpu.async_copy(u_hbm.at[idxR.at[i0 + 2]], g0, semA)

        pltpu.sync_copy(g1, acc.at[idxC.at[i1]], add=True)
        return 0

    lax.fori_loop(0, NP, _body, 0)
    plsc.subcore_barrier()
    _copy_out(acc, oacc, cid, sid)


_conv_pass = pl.kernel(
    _conv_body,
    out_type=jax.ShapeDtypeStruct((NC, N, H), _f32),
    mesh=_mesh,
    scratch_types=[
        pltpu.VMEM((NCH, CK), jnp.int32),
        pltpu.VMEM((NCH, CK), jnp.int32),
        pltpu.VMEM((CK, H), _f32),
        pltpu.VMEM((CK, H), _f32),
        pltpu.VMEM((208, H), _f32),
        pltpu.VMEM_SHARED((ND, H), _f32),
        pltpu.SemaphoreType.DMA,
        pltpu.SemaphoreType.DMA,
    ],
    compiler_params=_sc_params,
)


def _tc1_body(x_ref, wn, bn, we, be,
              aR0, aR1, aC0, aC1, cR0, cR1, cC0, cC1,
              u0_ref, dinv_ref):
    ea = aR0[...] + aR1[...] + aC0[...] + aC1[...]
    cr = cR0[:, 0:1] + cR1[:, 0:1]
    cc = cC0[:, 0:1] + cC1[:, 0:1]
    agg = jnp.dot(ea, we[...], preferred_element_type=_f32, precision=lax.Precision.HIGHEST) + (cr + cc) * be[...]
    dinv = lax.rsqrt(cc + 1.0)
    h = jnp.dot(x_ref[...], wn[...], preferred_element_type=_f32, precision=lax.Precision.HIGHEST) + bn[...] + agg
    u0_ref[...] = dinv * h
    dinv_ref[...] = dinv


def _full(shape):
    return pl.BlockSpec(shape, lambda i: (0,) * len(shape))


def _rows(width):
    return pl.BlockSpec((BR, width), lambda i: (i, 0))


_tc1 = pl.pallas_call(
    _tc1_body,
    grid=(NB,),
    in_specs=[
        _rows(DF), _full((DF, H)), _full((1, H)), _full((DE, H)), _full((1, H)),
        _rows(DE), _rows(DE), _rows(DE), _rows(DE),
        _rows(DE), _rows(DE), _rows(DE), _rows(DE),
    ],
    out_specs=[_rows(H), _rows(1)],
    out_shape=[
        jax.ShapeDtypeStruct((N, H), _f32),
        jax.ShapeDtypeStruct((N, 1), _f32),
    ],
)


def _tcmid_body(va, vb, u, dinv, wc, bc, x_ref, u_ref):
    s = va[...] + vb[...] + u[...]
    t = jnp.dot(s, wc[...], preferred_element_type=_f32, precision=lax.Precision.HIGHEST)
    xl = jnp.maximum(dinv[...] * t + bc[...], 0.0)
    x_ref[...] = xl
    u_ref[...] = dinv[...] * xl


_tcmid = pl.pallas_call(
    _tcmid_body,
    grid=(NB,),
    in_specs=[_rows(H), _rows(H), _rows(H), _rows(1), _full((H, H)), _full((1, H))],
    out_specs=[_rows(H), _rows(H)],
    out_shape=[
        jax.ShapeDtypeStruct((N, H), _f32),
        jax.ShapeDtypeStruct((N, H), _f32),
    ],
)


def _tcfin_body(va, vb, u, dinv, wc3, bc3, x1, x2, bat,
                wk1a, wk1b, wk1c, bk1, wk2r, bk2,
                out_ref, s1, s2, s3, cnts):
    i = pl.program_id(0)
    s = va[...] + vb[...] + u[...]
    x3 = jnp.maximum(
        dinv[...] * jnp.dot(s, wc3[...], preferred_element_type=_f32, precision=lax.Precision.HIGHEST) + bc3[...], 0.0)

    cols = lax.broadcasted_iota(jnp.int32, (BR, G), 1)
    oh = (bat[...] == cols).astype(_f32)

    @pl.when(i == 0)
    def _():
        s1[...] = jnp.zeros_like(s1)
        s2[...] = jnp.zeros_like(s2)
        s3[...] = jnp.zeros_like(s3)
        cnts[...] = jnp.zeros_like(cnts)

    dn = (((0,), (0,)), ((), ()))
    s1[...] += lax.dot_general(oh, x1[...], dn, preferred_element_type=_f32, precision=lax.Precision.HIGHEST)
    s2[...] += lax.dot_general(oh, x2[...], dn, preferred_element_type=_f32, precision=lax.Precision.HIGHEST)
    s3[...] += lax.dot_general(oh, x3, dn, preferred_element_type=_f32, precision=lax.Precision.HIGHEST)
    cnts[...] += lax.dot_general(oh, jnp.ones((BR, 1), _f32), dn,
                                 preferred_element_type=_f32, precision=lax.Precision.HIGHEST)

    @pl.when(i == NB - 1)
    def _():
        c = jnp.maximum(cnts[...], 1.0)
        z = (jnp.dot(s1[...] / c, wk1a[...], preferred_element_type=_f32, precision=lax.Precision.HIGHEST)
             + jnp.dot(s2[...] / c, wk1b[...], preferred_element_type=_f32, precision=lax.Precision.HIGHEST)
             + jnp.dot(s3[...] / c, wk1c[...], preferred_element_type=_f32, precision=lax.Precision.HIGHEST)
             + bk1[...])
        z = jnp.maximum(z, 0.0)
        o = jnp.sum(z * wk2r[...], axis=1)[None, :] + bk2[...]
        out_ref[...] = o


_tcfin = pl.pallas_call(
    _tcfin_body,
    grid=(NB,),
    in_specs=[
        _rows(H), _rows(H), _rows(H), _rows(1), _full((H, H)), _full((1, H)),
        _rows(H), _rows(H), _rows(1),
        _full((H, H // 2)), _full((H, H // 2)), _full((H, H // 2)),
        _full((1, H // 2)), _full((1, H // 2)), _full((1, 1)),
    ],
    out_specs=_full((1, G)),
    out_shape=jax.ShapeDtypeStruct((1, G), _f32),
    scratch_shapes=[
        pltpu.VMEM((G, H), _f32),
        pltpu.VMEM((G, H), _f32),
        pltpu.VMEM((G, H), _f32),
        pltpu.VMEM((G, 1), _f32),
    ],
)


def kernel(x, edge_index, edge_attr, batch,
           Wn, bn, We, be, Wc1, bc1, Wc2, bc2, Wc3, bc3, Wk1, bk1, Wk2, bk2):
    row2 = edge_index[0].reshape(NW, EPW)
    col2 = edge_index[1].reshape(NW, EPW)
    rowg = jnp.pad(row2, ((0, 0), (0, PAD))).reshape(NW, NCH, CK)
    rows = jnp.pad(row2, ((0, 0), (0, PAD)), constant_values=N).reshape(NW, NCH, CK)
    cols = jnp.pad(col2, ((0, 0), (0, PAD)), constant_values=N).reshape(NW, NCH, CK)

    aR, aC, cR, cC = _edge_pass(edge_attr, rows, cols)
    u0, dinv = _tc1(x, Wn, bn.reshape(1, H), We, be.reshape(1, H),
                    aR[0], aR[1], aC[0], aC[1], cR[0], cR[1], cC[0], cC[1])

    v = _conv_pass(u0, rowg, cols)
    x1, u1 = _tcmid(v[0], v[1], u0, dinv, Wc1, bc1.reshape(1, H))
    v = _conv_pass(u1, rowg, cols)
    x2, u2 = _tcmid(v[0], v[1], u1, dinv, Wc2, bc2.reshape(1, H))
    v = _conv_pass(u2, rowg, cols)

    out = _tcfin(v[0], v[1], u2, dinv, Wc3, bc3.reshape(1, H),
                 x1, x2, batch.reshape(N, 1),
                 Wk1[0:H], Wk1[H:2 * H], Wk1[2 * H:3 * H],
                 bk1.reshape(1, H // 2), Wk2.reshape(1, H // 2),
                 bk2.reshape(1, 1))
    return out.reshape(G)
